# baseline (speedup 1.0000x reference)
import functools

import jax
import jax.numpy as jnp
from jax import lax
from jax.experimental import pallas as pl
from jax.experimental.pallas import tpu as pltpu
from jax.experimental.pallas import tpu_sc as plsc

N = 10000
E = 320000
D = 128
LANES = 16
NCORES = 2
NSUB = 16
NWORK = NCORES * NSUB

EDGES_PER_TILE = E // NWORK
CHUNK = 125
NCHUNK = EDGES_PER_TILE // CHUNK
RCHUNK = 80
NRCHUNK = N // RCHUNK

_mesh = plsc.VectorSubcoreMesh(core_axis_name="c", subcore_axis_name="s")


def _matmul(features, w):
    def body(x_ref, w_ref, o_ref):
        o_ref[...] = jnp.dot(x_ref[...], w_ref[...],
                             preferred_element_type=jnp.float32)

    return pl.pallas_call(
        body,
        grid=(10,),
        in_specs=[
            pl.BlockSpec((N // 10, D), lambda i: (i, 0)),
            pl.BlockSpec((D, D), lambda i: (0, 0)),
        ],
        out_specs=pl.BlockSpec((N // 10, D), lambda i: (i, 0)),
        out_shape=jax.ShapeDtypeStruct((N, D), jnp.float32),
    )(features, w)


@functools.partial(
    pl.kernel,
    out_type=jax.ShapeDtypeStruct((NCORES, N, D), jnp.float32),
    mesh=_mesh,
    scratch_types=[
        pltpu.VMEM_SHARED((N, D), jnp.float32),
        pltpu.VMEM((4, CHUNK), jnp.int32),
        pltpu.VMEM((4, CHUNK), jnp.int32),
        pltpu.VMEM((4, CHUNK), jnp.float32),
        pltpu.VMEM((CHUNK, D), jnp.float32),
        pltpu.VMEM((CHUNK, D), jnp.float32),
        pltpu.VMEM((CHUNK, D), jnp.float32),
        [pltpu.SemaphoreType.DMA] * 3,
        [pltpu.SemaphoreType.DMA] * 3,
        [pltpu.SemaphoreType.DMA] * 4,
        [pltpu.SemaphoreType.DMA] * 4,
    ],
    compiler_params=pltpu.CompilerParams(needs_layout_passes=False),
)
def _spmm_round(base_hbm, col_hbm, row_hbm, val_hbm, out_hbm,
                acc, colb, rowb, valb, rbuf0, rbuf1, rbuf2,
                gsems, ssems, csems, rvsems):
    c = lax.axis_index("c")
    s = lax.axis_index("s")
    wid = c * NSUB + s
    bufs = (rbuf0, rbuf1, rbuf2)

    def stage(gch, t):
        pltpu.async_copy(col_hbm.at[wid, gch], colb.at[t], csems[t])
        pltpu.async_copy(row_hbm.at[wid, gch], rowb.at[t], rvsems[t])
        pltpu.async_copy(val_hbm.at[wid, gch], valb.at[t], rvsems[t])

    def wait_col(gch, t):
        pltpu.make_async_copy(col_hbm.at[wid, gch], colb.at[t],
                              csems[t]).wait()

    def wait_rv(gch, t):
        pltpu.make_async_copy(row_hbm.at[wid, gch], rowb.at[t],
                              rvsems[t]).wait()
        pltpu.make_async_copy(val_hbm.at[wid, gch], valb.at[t],
                              rvsems[t]).wait()

    def issue_gather(r, t):
        pltpu.async_copy(base_hbm.at[colb.at[t]], bufs[r], gsems[r])

    def wait_gather(r, t):
        pltpu.make_async_copy(base_hbm.at[colb.at[t]], bufs[r],
                              gsems[r]).wait()

    def wait_scatter(r, t):
        pltpu.make_async_copy(bufs[r], acc.at[rowb.at[t]], ssems[r]).wait()

    for t in range(3):
        stage(t, t)

    zero16 = jnp.zeros((LANES,), jnp.float32)

    def zbody(r, carry):
        for j in range(D // LANES):
            rbuf0.at[r, pl.ds(j * LANES, LANES)][...] = zero16
        return carry

    lax.fori_loop(0, RCHUNK, zbody, 0)
    zsrc = rbuf0.at[pl.ds(0, RCHUNK)]
    for i in range((NRCHUNK + NSUB - 1) // NSUB):
        cid = s + NSUB * i

        @pl.when(cid < NRCHUNK)
        def _():
            r0 = pl.multiple_of(cid * RCHUNK, 8)
            pltpu.sync_copy(zsrc, acc.at[pl.ds(r0, RCHUNK)])

    plsc.subcore_barrier()

    wait_col(0, 0)
    issue_gather(0, 0)
    wait_col(1, 1)
    issue_gather(1, 1)

    UNROLL = 12

    def slot(g, u):
        r = u % 3
        t = u % 4

        @pl.when(g < NCHUNK)
        def _():
            wait_gather(r, t)
            wait_rv(g, t)
            vref = valb.at[t]

            @plsc.parallel_loop(0, CHUNK, unroll=10)
            def mul_body(k):
                vv = plsc.load_gather(vref,
                                      [jnp.full((LANES,), k, jnp.int32)])
                for j in range(D // LANES):
                    rr = bufs[r].at[k, pl.ds(j * LANES, LANES)]
                    rr[...] = rr[...] * vv

            @pl.when(g >= 1)
            def _():
                wait_scatter((u - 1) % 3, (u - 1) % 4)

            pltpu.async_copy(bufs[r], acc.at[rowb.at[t]], ssems[r],
                             add=True)

            @pl.when(g + 3 < NCHUNK)
            def _():
                stage(g + 3, (u + 3) % 4)

            @pl.when(g + 2 < NCHUNK)
            def _():
                wait_col(g + 2, (u + 2) % 4)
                issue_gather((u + 2) % 3, (u + 2) % 4)

    def loop_body(i, carry):
        for u in range(UNROLL):
            slot(UNROLL * i + u, u)
        return carry

    niter = (NCHUNK + UNROLL - 1) // UNROLL
    lax.fori_loop(0, niter, loop_body, 0)
    u_last = (NCHUNK - 1) % UNROLL
    wait_scatter(u_last % 3, u_last % 4)
    plsc.subcore_barrier()
    for i in range((NRCHUNK + NSUB - 1) // NSUB):
        cid = s + NSUB * i

        @pl.when(cid < NRCHUNK)
        def _():
            r0 = pl.multiple_of(cid * RCHUNK, 8)
            pltpu.sync_copy(acc.at[pl.ds(r0, RCHUNK)],
                            out_hbm.at[c, pl.ds(r0, RCHUNK)])


def _reduce_bias(parts, bias):
    def body(p_ref, b_ref, o_ref):
        o_ref[...] = p_ref[0] + p_ref[1] + b_ref[...]

    return pl.pallas_call(
        body,
        grid=(10,),
        in_specs=[
            pl.BlockSpec((2, N // 10, D), lambda i: (0, i, 0)),
            pl.BlockSpec((1, D), lambda i: (0, 0)),
        ],
        out_specs=pl.BlockSpec((N // 10, D), lambda i: (i, 0)),
        out_shape=jax.ShapeDtypeStruct((N, D), jnp.float32),
    )(parts, bias)


def kernel(adj_indices, adj_values, features, W, b):
    row3 = adj_indices[0].reshape(NWORK, NCHUNK, CHUNK)
    col3 = adj_indices[1].reshape(NWORK, NCHUNK, CHUNK)
    val3 = adj_values.reshape(NWORK, NCHUNK, CHUNK)
    base = _matmul(features, W)
    zero_bias = jnp.zeros_like(b)
    for it in range(2):
        parts = _spmm_round(base, col3, row3, val3)
        base = _reduce_bias(parts, b if it == 1 else zero_bias)
    return base


# speedup vs baseline: 1.0396x; 1.0396x over previous
import functools

import jax
import jax.numpy as jnp
from jax import lax
from jax.experimental import pallas as pl
from jax.experimental.pallas import tpu as pltpu
from jax.experimental.pallas import tpu_sc as plsc

N = 10000
E = 320000
D = 128
LANES = 16
NCORES = 2
NSUB = 16
NWORK = NCORES * NSUB

EDGES_PER_TILE = E // NWORK
CHUNK = 125
NCHUNK = EDGES_PER_TILE // CHUNK
RCHUNK = 80
NRCHUNK = N // RCHUNK

_mesh = plsc.VectorSubcoreMesh(core_axis_name="c", subcore_axis_name="s")


def _matmul(features, w):
    def body(x_ref, w_ref, o_ref):
        o_ref[...] = jnp.dot(x_ref[...], w_ref[...],
                             preferred_element_type=jnp.float32)

    return pl.pallas_call(
        body,
        grid=(10,),
        in_specs=[
            pl.BlockSpec((N // 10, D), lambda i: (i, 0)),
            pl.BlockSpec((D, D), lambda i: (0, 0)),
        ],
        out_specs=pl.BlockSpec((N // 10, D), lambda i: (i, 0)),
        out_shape=jax.ShapeDtypeStruct((N, D), jnp.float32),
    )(features, w)


@functools.partial(
    pl.kernel,
    out_type=jax.ShapeDtypeStruct((NCORES, N, D), jnp.float32),
    mesh=_mesh,
    scratch_types=[
        pltpu.VMEM_SHARED((N, D), jnp.float32),
        pltpu.VMEM((4, CHUNK), jnp.int32),
        pltpu.VMEM((4, CHUNK), jnp.int32),
        pltpu.VMEM((4, CHUNK), jnp.float32),
        pltpu.VMEM((CHUNK, D), jnp.float32),
        pltpu.VMEM((CHUNK, D), jnp.float32),
        pltpu.VMEM((CHUNK, D), jnp.float32),
        [pltpu.SemaphoreType.DMA] * 3,
        [pltpu.SemaphoreType.DMA] * 3,
        [pltpu.SemaphoreType.DMA] * 4,
        [pltpu.SemaphoreType.DMA] * 4,
    ],
    compiler_params=pltpu.CompilerParams(needs_layout_passes=False),
)
def _spmm_round(base_hbm, col_hbm, row_hbm, val_hbm, out_hbm,
                acc, colb, rowb, valb, rbuf0, rbuf1, rbuf2,
                gsems, ssems, csems, rvsems):
    c = lax.axis_index("c")
    s = lax.axis_index("s")
    wid = c * NSUB + s
    bufs = (rbuf0, rbuf1, rbuf2)

    def stage(gch, t):
        pltpu.async_copy(col_hbm.at[wid, gch], colb.at[t], csems[t])
        pltpu.async_copy(row_hbm.at[wid, gch], rowb.at[t], rvsems[t])
        pltpu.async_copy(val_hbm.at[wid, gch], valb.at[t], rvsems[t])

    def wait_col(gch, t):
        pltpu.make_async_copy(col_hbm.at[wid, gch], colb.at[t],
                              csems[t]).wait()

    def wait_rv(gch, t):
        pltpu.make_async_copy(row_hbm.at[wid, gch], rowb.at[t],
                              rvsems[t]).wait()
        pltpu.make_async_copy(val_hbm.at[wid, gch], valb.at[t],
                              rvsems[t]).wait()

    def issue_gather(r, t):
        pltpu.async_copy(base_hbm.at[colb.at[t]], bufs[r], gsems[r])

    def wait_gather(r, t):
        pltpu.make_async_copy(base_hbm.at[colb.at[t]], bufs[r],
                              gsems[r]).wait()

    def wait_scatter(r, t):
        pltpu.make_async_copy(bufs[r], acc.at[rowb.at[t]], ssems[r]).wait()

    for t in range(3):
        stage(t, t)

    zero16 = jnp.zeros((LANES,), jnp.float32)

    def zbody(r, carry):
        for j in range(D // LANES):
            rbuf0.at[r, pl.ds(j * LANES, LANES)][...] = zero16
        return carry

    lax.fori_loop(0, RCHUNK, zbody, 0)
    zsrc = rbuf0.at[pl.ds(0, RCHUNK)]
    for i in range((NRCHUNK + NSUB - 1) // NSUB):
        cid = s + NSUB * i

        @pl.when(cid < NRCHUNK)
        def _():
            r0 = pl.multiple_of(cid * RCHUNK, 8)
            pltpu.sync_copy(zsrc, acc.at[pl.ds(r0, RCHUNK)])

    plsc.subcore_barrier()

    wait_col(0, 0)
    issue_gather(0, 0)
    wait_col(1, 1)
    issue_gather(1, 1)

    UNROLL = 12

    def slot(g, u):
        r = u % 3
        t = u % 4

        @pl.when(g < NCHUNK)
        def _():
            wait_gather(r, t)
            wait_rv(g, t)
            vref = valb.at[t]

            @plsc.parallel_loop(0, CHUNK, unroll=5)
            def mul_body(k):
                vv = plsc.load_gather(vref,
                                      [jnp.full((LANES,), k, jnp.int32)])
                for j in range(D // LANES):
                    rr = bufs[r].at[k, pl.ds(j * LANES, LANES)]
                    rr[...] = rr[...] * vv

            @pl.when(g >= 1)
            def _():
                wait_scatter((u - 1) % 3, (u - 1) % 4)

            pltpu.async_copy(bufs[r], acc.at[rowb.at[t]], ssems[r],
                             add=True)

            @pl.when(g + 3 < NCHUNK)
            def _():
                stage(g + 3, (u + 3) % 4)

            @pl.when(g + 2 < NCHUNK)
            def _():
                wait_col(g + 2, (u + 2) % 4)
                issue_gather((u + 2) % 3, (u + 2) % 4)

    def loop_body(i, carry):
        for u in range(UNROLL):
            slot(UNROLL * i + u, u)
        return carry

    niter = (NCHUNK + UNROLL - 1) // UNROLL
    lax.fori_loop(0, niter, loop_body, 0)
    u_last = (NCHUNK - 1) % UNROLL
    wait_scatter(u_last % 3, u_last % 4)
    plsc.subcore_barrier()
    for i in range((NRCHUNK + NSUB - 1) // NSUB):
        cid = s + NSUB * i

        @pl.when(cid < NRCHUNK)
        def _():
            r0 = pl.multiple_of(cid * RCHUNK, 8)
            pltpu.sync_copy(acc.at[pl.ds(r0, RCHUNK)],
                            out_hbm.at[c, pl.ds(r0, RCHUNK)])


def _reduce_bias(parts, bias):
    def body(p_ref, b_ref, o_ref):
        o_ref[...] = p_ref[0] + p_ref[1] + b_ref[...]

    return pl.pallas_call(
        body,
        grid=(10,),
        in_specs=[
            pl.BlockSpec((2, N // 10, D), lambda i: (0, i, 0)),
            pl.BlockSpec((1, D), lambda i: (0, 0)),
        ],
        out_specs=pl.BlockSpec((N // 10, D), lambda i: (i, 0)),
        out_shape=jax.ShapeDtypeStruct((N, D), jnp.float32),
    )(parts, bias)


def kernel(adj_indices, adj_values, features, W, b):
    row3 = adj_indices[0].reshape(NWORK, NCHUNK, CHUNK)
    col3 = adj_indices[1].reshape(NWORK, NCHUNK, CHUNK)
    val3 = adj_values.reshape(NWORK, NCHUNK, CHUNK)
    base = _matmul(features, W)
    zero_bias = jnp.zeros_like(b)
    for it in range(2):
        parts = _spmm_round(base, col3, row3, val3)
        base = _reduce_bias(parts, b if it == 1 else zero_bias)
    return base
